# baseline (speedup 1.0000x reference)
import functools

import jax
import jax.numpy as jnp
from jax import lax
from jax.experimental import pallas as pl
from jax.experimental.pallas import tpu as pltpu
from jax.experimental.pallas import tpu_sc as plsc

_B, _D = 16384, 4096
_SD = 2048
_SB = _B * _D // _SD
_NW = 32
_SUBROWS_PER_W = _SB // _NW
_CHUNK = 16
_NCHUNK = _SUBROWS_PER_W // _CHUNK
_IDX_PAD = 416


def _sc_body(x_hbm, idx_hbm, out_hbm, idx_v, buf, sem_in, sem_out):
    c = lax.axis_index("c")
    s = lax.axis_index("s")
    wid = s * 2 + c
    base = wid * _SUBROWS_PER_W
    pltpu.sync_copy(idx_hbm, idx_v)
    lane = lax.iota(jnp.int32, 16)

    def chunk_body(kk, carry):
        rvec = lane + (base + kk * _CHUNK)
        pltpu.async_copy(x_hbm.at[rvec], buf, sem_in).wait()
        pltpu.async_copy(buf, out_hbm.at[rvec], sem_out).wait()
        return carry

    lax.fori_loop(0, _NCHUNK, chunk_body, 0)


def kernel(x, mask_indices):
    idx = mask_indices.astype(jnp.int32)
    n = idx.shape[0]
    idx = jnp.pad(idx, (0, _IDX_PAD - n), mode="edge")

    mesh = plsc.VectorSubcoreMesh(core_axis_name="c", subcore_axis_name="s")
    run = functools.partial(
        pl.kernel,
        mesh=mesh,
        out_type=jax.ShapeDtypeStruct((_SB, _SD), jnp.float32),
        scratch_types=[
            pltpu.VMEM((_IDX_PAD,), jnp.int32),
            pltpu.VMEM((_CHUNK, _SD), jnp.float32),
            pltpu.SemaphoreType.DMA,
            pltpu.SemaphoreType.DMA,
        ],
        compiler_params=pltpu.CompilerParams(needs_layout_passes=False),
    )(_sc_body)
    return run(x.reshape(_SB, _SD), idx).reshape(_B, _D)


# speedup vs baseline: 4.2486x; 4.2486x over previous
import functools

import jax
import jax.numpy as jnp
from jax import lax
from jax.experimental import pallas as pl
from jax.experimental.pallas import tpu as pltpu
from jax.experimental.pallas import tpu_sc as plsc

_B, _D = 16384, 4096
_BLOCK_ROWS = 864
_IDX_PAD = 416
_NJ = _IDX_PAD // 16
_FILL = _D // 16


def _sc_mask_body(idx_hbm, mask_hbm, idx_v, mask_v, sem):
    c = lax.axis_index("c")
    s = lax.axis_index("s")
    wid = s * 2 + c

    @pl.when(wid == 0)
    def _():
        pltpu.sync_copy(idx_hbm, idx_v)
        ones = jnp.ones((16,), jnp.float32)
        zeros = jnp.zeros((16,), jnp.float32)

        def fill_body(i, carry):
            mask_v[pl.ds(i * 16, 16)] = ones
            return carry

        lax.fori_loop(0, _FILL, fill_body, 0)

        def j_body(j, carry):
            colv = idx_v[pl.ds(j * 16, 16)]
            plsc.store_scatter(mask_v, [colv], zeros)
            return carry

        lax.fori_loop(0, _NJ, j_body, 0)
        pltpu.sync_copy(mask_v, mask_hbm)


def _tc_body(mask_ref, x_ref, o_ref):
    o_ref[...] = x_ref[...] * mask_ref[...]


def kernel(x, mask_indices):
    idx = mask_indices.astype(jnp.int32)
    n = idx.shape[0]
    idx = jnp.pad(idx, (0, _IDX_PAD - n), mode="edge")

    mesh = plsc.VectorSubcoreMesh(core_axis_name="c", subcore_axis_name="s")
    build_mask = functools.partial(
        pl.kernel,
        mesh=mesh,
        out_type=jax.ShapeDtypeStruct((_D,), jnp.float32),
        scratch_types=[
            pltpu.VMEM((_IDX_PAD,), jnp.int32),
            pltpu.VMEM((_D,), jnp.float32),
            pltpu.SemaphoreType.DMA,
        ],
        compiler_params=pltpu.CompilerParams(needs_layout_passes=False),
    )(_sc_mask_body)
    mask = build_mask(idx).reshape(1, _D)

    grid = (pl.cdiv(_B, _BLOCK_ROWS),)
    return pl.pallas_call(
        _tc_body,
        grid=grid,
        in_specs=[
            pl.BlockSpec((1, _D), lambda i: (0, 0)),
            pl.BlockSpec((_BLOCK_ROWS, _D), lambda i: (i, 0)),
        ],
        out_specs=pl.BlockSpec((_BLOCK_ROWS, _D), lambda i: (i, 0)),
        out_shape=jax.ShapeDtypeStruct((_B, _D), jnp.float32),
        compiler_params=pltpu.CompilerParams(
            dimension_semantics=("arbitrary",),
        ),
    )(mask, x)


# speedup vs baseline: 4.2804x; 1.0075x over previous
import functools

import jax
import jax.numpy as jnp
from jax import lax
from jax.experimental import pallas as pl
from jax.experimental.pallas import tpu as pltpu
from jax.experimental.pallas import tpu_sc as plsc

_B, _D = 16384, 4096
_BLOCK_ROWS = 512
_HALF_BLKS = _B // _BLOCK_ROWS // 2
_IDX_PAD = 416
_NJ = _IDX_PAD // 16
_FILL = _D // 16
_TC_IDX_PAD = 512


def _sc_mask_body(idx_hbm, mask_hbm, idx_v, mask_v, sem):
    c = lax.axis_index("c")
    s = lax.axis_index("s")
    wid = s * 2 + c

    @pl.when(wid == 0)
    def _():
        pltpu.sync_copy(idx_hbm, idx_v)
        ones = jnp.ones((16,), jnp.float32)
        zeros = jnp.zeros((16,), jnp.float32)

        def fill_body(i, carry):
            mask_v[pl.ds(i * 16, 16)] = ones
            return carry

        lax.fori_loop(0, _FILL, fill_body, 0)

        def j_body(j, carry):
            colv = idx_v[pl.ds(j * 16, 16)]
            plsc.store_scatter(mask_v, [colv], zeros)
            return carry

        lax.fori_loop(0, _NJ, j_body, 0)
        pltpu.sync_copy(mask_v, mask_hbm)


def _tc_top_body(idx_ref, x_ref, o_ref, mask_ref):
    @pl.when(pl.program_id(0) == 0)
    def _():
        cols = jax.lax.broadcasted_iota(jnp.int32, (1, _D), 1)
        idx = idx_ref[...].reshape(_TC_IDX_PAD, 1)
        hit = jnp.any(idx == cols, axis=0, keepdims=True)
        mask_ref[...] = jnp.where(hit, 0.0, 1.0)

    o_ref[...] = x_ref[...] * mask_ref[...]


def _tc_bot_body(mask_ref, x_ref, prev_ref, o_ref):
    del prev_ref
    o_ref[...] = x_ref[...] * mask_ref[...]


def kernel(x, mask_indices):
    idx = mask_indices.astype(jnp.int32)
    n = idx.shape[0]
    idx_sc = jnp.pad(idx, (0, _IDX_PAD - n), mode="edge")
    idx_tc = jnp.pad(idx, (0, _TC_IDX_PAD - n),
                     constant_values=_D).reshape(1, _TC_IDX_PAD)

    mesh = plsc.VectorSubcoreMesh(core_axis_name="c", subcore_axis_name="s")
    build_mask = functools.partial(
        pl.kernel,
        mesh=mesh,
        out_type=jax.ShapeDtypeStruct((_D,), jnp.float32),
        scratch_types=[
            pltpu.VMEM((_IDX_PAD,), jnp.int32),
            pltpu.VMEM((_D,), jnp.float32),
            pltpu.SemaphoreType.DMA,
        ],
        compiler_params=pltpu.CompilerParams(needs_layout_passes=False),
    )(_sc_mask_body)
    sc_mask = build_mask(idx_sc).reshape(1, _D)

    out_top = pl.pallas_call(
        _tc_top_body,
        grid=(_HALF_BLKS,),
        in_specs=[
            pl.BlockSpec((1, _TC_IDX_PAD), lambda i: (0, 0)),
            pl.BlockSpec((_BLOCK_ROWS, _D), lambda i: (i, 0)),
        ],
        out_specs=pl.BlockSpec((_BLOCK_ROWS, _D), lambda i: (i, 0)),
        out_shape=jax.ShapeDtypeStruct((_B, _D), jnp.float32),
        scratch_shapes=[pltpu.VMEM((1, _D), jnp.float32)],
        compiler_params=pltpu.CompilerParams(
            dimension_semantics=("arbitrary",),
        ),
    )(idx_tc, x)

    return pl.pallas_call(
        _tc_bot_body,
        grid=(_HALF_BLKS,),
        in_specs=[
            pl.BlockSpec((1, _D), lambda i: (0, 0)),
            pl.BlockSpec((_BLOCK_ROWS, _D), lambda i: (i + _HALF_BLKS, 0)),
            pl.BlockSpec(memory_space=pltpu.MemorySpace.HBM),
        ],
        out_specs=pl.BlockSpec((_BLOCK_ROWS, _D), lambda i: (i + _HALF_BLKS, 0)),
        out_shape=jax.ShapeDtypeStruct((_B, _D), jnp.float32),
        input_output_aliases={2: 0},
        compiler_params=pltpu.CompilerParams(
            dimension_semantics=("arbitrary",),
        ),
    )(sc_mask, x, out_top)


# speedup vs baseline: 4.3105x; 1.0070x over previous
import functools

import jax
import jax.numpy as jnp
from jax import lax
from jax.experimental import pallas as pl
from jax.experimental.pallas import tpu as pltpu
from jax.experimental.pallas import tpu_sc as plsc

_B, _D = 16384, 4096
_BLOCK_ROWS = 512
_HALF_BLKS = _B // _BLOCK_ROWS // 2
_IDX_PAD = 416
_NJ = _IDX_PAD // 16
_FILL = _D // 16
_TC_IDX_PAD = 512


def _sc_mask_body(idx_hbm, mask_hbm, idx_v, mask_v, sem):
    c = lax.axis_index("c")
    s = lax.axis_index("s")
    wid = s * 2 + c

    @pl.when(wid == 0)
    def _():
        pltpu.sync_copy(idx_hbm, idx_v)
        ones = jnp.ones((16,), jnp.float32)
        zeros = jnp.zeros((16,), jnp.float32)

        def fill_body(i, carry):
            mask_v[pl.ds(i * 16, 16)] = ones
            return carry

        lax.fori_loop(0, _FILL, fill_body, 0)

        def j_body(j, carry):
            colv = idx_v[pl.ds(j * 16, 16)]
            plsc.store_scatter(mask_v, [colv], zeros)
            return carry

        lax.fori_loop(0, _NJ, j_body, 0)
        pltpu.sync_copy(mask_v, mask_hbm)


def _tc_top_body(idx_ref, x_ref, o_ref, mask_ref):
    @pl.when(pl.program_id(0) == 0)
    def _():
        cols = jax.lax.broadcasted_iota(jnp.int32, (1, _D), 1)
        idx = idx_ref[...].reshape(_TC_IDX_PAD, 1)
        hit = jnp.any(idx == cols, axis=0, keepdims=True)
        mask_ref[...] = jnp.where(hit, 0.0, 1.0)

    o_ref[...] = x_ref[...] * mask_ref[...]


def _tc_bot_body(mask_ref, x_ref, prev_ref, o_ref):
    del prev_ref
    o_ref[...] = x_ref[...] * mask_ref[...]


def kernel(x, mask_indices):
    idx = mask_indices.astype(jnp.int32)
    n = idx.shape[0]
    idx_sc = jnp.pad(idx, (0, _IDX_PAD - n), mode="edge")
    idx_tc = jnp.pad(idx, (0, _TC_IDX_PAD - n),
                     constant_values=_D).reshape(1, _TC_IDX_PAD)

    mesh = plsc.VectorSubcoreMesh(
        core_axis_name="c", subcore_axis_name="s", num_cores=1)
    build_mask = functools.partial(
        pl.kernel,
        mesh=mesh,
        out_type=jax.ShapeDtypeStruct((_D,), jnp.float32),
        scratch_types=[
            pltpu.VMEM((_IDX_PAD,), jnp.int32),
            pltpu.VMEM((_D,), jnp.float32),
            pltpu.SemaphoreType.DMA,
        ],
        compiler_params=pltpu.CompilerParams(needs_layout_passes=False),
    )(_sc_mask_body)
    sc_mask = build_mask(idx_sc).reshape(1, _D)

    out_top = pl.pallas_call(
        _tc_top_body,
        grid=(_HALF_BLKS,),
        in_specs=[
            pl.BlockSpec((1, _TC_IDX_PAD), lambda i: (0, 0)),
            pl.BlockSpec((_BLOCK_ROWS, _D), lambda i: (i, 0)),
        ],
        out_specs=pl.BlockSpec((_BLOCK_ROWS, _D), lambda i: (i, 0)),
        out_shape=jax.ShapeDtypeStruct((_B, _D), jnp.float32),
        scratch_shapes=[pltpu.VMEM((1, _D), jnp.float32)],
        compiler_params=pltpu.CompilerParams(
            dimension_semantics=("arbitrary",),
        ),
    )(idx_tc, x)

    return pl.pallas_call(
        _tc_bot_body,
        grid=(_HALF_BLKS,),
        in_specs=[
            pl.BlockSpec((1, _D), lambda i: (0, 0)),
            pl.BlockSpec((_BLOCK_ROWS, _D), lambda i: (i + _HALF_BLKS, 0)),
            pl.BlockSpec(memory_space=pltpu.MemorySpace.HBM),
        ],
        out_specs=pl.BlockSpec((_BLOCK_ROWS, _D), lambda i: (i + _HALF_BLKS, 0)),
        out_shape=jax.ShapeDtypeStruct((_B, _D), jnp.float32),
        input_output_aliases={2: 0},
        compiler_params=pltpu.CompilerParams(
            dimension_semantics=("arbitrary",),
        ),
    )(sc_mask, x, out_top)


# speedup vs baseline: 4.3105x; 1.0000x over previous
import functools

import jax
import jax.numpy as jnp
from jax import lax
from jax.experimental import pallas as pl
from jax.experimental.pallas import tpu as pltpu
from jax.experimental.pallas import tpu_sc as plsc

_B, _D = 16384, 4096
_BLOCK_ROWS = 512
_HALF_BLKS = _B // _BLOCK_ROWS // 2
_IDX_PAD = 416
_NJ = _IDX_PAD // 16
_FILL = _D // 16
_TC_IDX_PAD = 512


def _sc_mask_body(idx_hbm, mask_hbm, idx_v, mask_v, sem):
    c = lax.axis_index("c")
    s = lax.axis_index("s")
    wid = s * 2 + c

    @pl.when(wid == 0)
    def _():
        pltpu.sync_copy(idx_hbm, idx_v)
        ones = jnp.ones((16,), jnp.float32)
        zeros = jnp.zeros((16,), jnp.float32)

        def fill_body(i, carry):
            mask_v[pl.ds(i * 16, 16)] = ones
            return carry

        lax.fori_loop(0, _FILL, fill_body, 0)

        def j_body(j, carry):
            colv = idx_v[pl.ds(j * 16, 16)]
            plsc.store_scatter(mask_v, [colv], zeros)
            return carry

        lax.fori_loop(0, _NJ, j_body, 0)
        pltpu.sync_copy(mask_v, mask_hbm)


def _tc_top_body(idx_ref, x_ref, o_ref, mask_ref):
    @pl.when(pl.program_id(0) == 0)
    def _():
        cols = jax.lax.broadcasted_iota(jnp.int32, (1, _D), 1)
        idx = idx_ref[...].reshape(_TC_IDX_PAD, 1)
        hit = jnp.any(idx == cols, axis=0, keepdims=True)
        mask_ref[...] = jnp.where(hit, 0.0, 1.0)

    o_ref[...] = x_ref[...] * mask_ref[...]


def _tc_bot_body(mask_ref, x_ref, prev_ref, o_ref):
    del prev_ref
    o_ref[...] = x_ref[...] * mask_ref[...]


def kernel(x, mask_indices):
    idx = mask_indices.astype(jnp.int32)
    n = idx.shape[0]
    idx_sc = jnp.pad(idx, (0, _IDX_PAD - n), mode="edge")
    idx_tc = jnp.pad(idx, (0, _TC_IDX_PAD - n),
                     constant_values=_D).reshape(1, _TC_IDX_PAD)

    mesh = plsc.VectorSubcoreMesh(
        core_axis_name="c", subcore_axis_name="s",
        num_cores=1, num_subcores=1)
    build_mask = functools.partial(
        pl.kernel,
        mesh=mesh,
        out_type=jax.ShapeDtypeStruct((_D,), jnp.float32),
        scratch_types=[
            pltpu.VMEM((_IDX_PAD,), jnp.int32),
            pltpu.VMEM((_D,), jnp.float32),
            pltpu.SemaphoreType.DMA,
        ],
        compiler_params=pltpu.CompilerParams(needs_layout_passes=False),
    )(_sc_mask_body)
    sc_mask = build_mask(idx_sc).reshape(1, _D)

    out_top = pl.pallas_call(
        _tc_top_body,
        grid=(_HALF_BLKS,),
        in_specs=[
            pl.BlockSpec((1, _TC_IDX_PAD), lambda i: (0, 0)),
            pl.BlockSpec((_BLOCK_ROWS, _D), lambda i: (i, 0)),
        ],
        out_specs=pl.BlockSpec((_BLOCK_ROWS, _D), lambda i: (i, 0)),
        out_shape=jax.ShapeDtypeStruct((_B, _D), jnp.float32),
        scratch_shapes=[pltpu.VMEM((1, _D), jnp.float32)],
        compiler_params=pltpu.CompilerParams(
            dimension_semantics=("arbitrary",),
        ),
    )(idx_tc, x)

    return pl.pallas_call(
        _tc_bot_body,
        grid=(_HALF_BLKS,),
        in_specs=[
            pl.BlockSpec((1, _D), lambda i: (0, 0)),
            pl.BlockSpec((_BLOCK_ROWS, _D), lambda i: (i + _HALF_BLKS, 0)),
            pl.BlockSpec(memory_space=pltpu.MemorySpace.HBM),
        ],
        out_specs=pl.BlockSpec((_BLOCK_ROWS, _D), lambda i: (i + _HALF_BLKS, 0)),
        out_shape=jax.ShapeDtypeStruct((_B, _D), jnp.float32),
        input_output_aliases={2: 0},
        compiler_params=pltpu.CompilerParams(
            dimension_semantics=("arbitrary",),
        ),
    )(sc_mask, x, out_top)


# speedup vs baseline: 4.3674x; 1.0132x over previous
import functools

import jax
import jax.numpy as jnp
from jax import lax
from jax.experimental import pallas as pl
from jax.experimental.pallas import tpu as pltpu
from jax.experimental.pallas import tpu_sc as plsc

_B, _D = 16384, 4096
_BLOCK_ROWS = 864
_TOP_BLKS = 10
_BOT_BLKS = 9
_IDX_PAD = 416
_NJ = _IDX_PAD // 16
_FILL = _D // 16
_TC_IDX_PAD = 512


def _sc_mask_body(idx_hbm, mask_hbm, idx_v, mask_v, sem):
    c = lax.axis_index("c")
    s = lax.axis_index("s")
    wid = s * 2 + c

    @pl.when(wid == 0)
    def _():
        pltpu.sync_copy(idx_hbm, idx_v)
        ones = jnp.ones((16,), jnp.float32)
        zeros = jnp.zeros((16,), jnp.float32)

        def fill_body(i, carry):
            mask_v[pl.ds(i * 16, 16)] = ones
            return carry

        lax.fori_loop(0, _FILL, fill_body, 0)

        def j_body(j, carry):
            colv = idx_v[pl.ds(j * 16, 16)]
            plsc.store_scatter(mask_v, [colv], zeros)
            return carry

        lax.fori_loop(0, _NJ, j_body, 0)
        pltpu.sync_copy(mask_v, mask_hbm)


def _tc_top_body(idx_ref, x_ref, o_ref, mask_ref):
    @pl.when(pl.program_id(0) == 0)
    def _():
        cols = jax.lax.broadcasted_iota(jnp.int32, (1, _D), 1)
        idx = idx_ref[...].reshape(_TC_IDX_PAD, 1)
        hit = jnp.any(idx == cols, axis=0, keepdims=True)
        mask_ref[...] = jnp.where(hit, 0.0, 1.0)

    o_ref[...] = x_ref[...] * mask_ref[...]


def _tc_bot_body(mask_ref, x_ref, prev_ref, o_ref):
    del prev_ref
    o_ref[...] = x_ref[...] * mask_ref[...]


def kernel(x, mask_indices):
    idx = mask_indices.astype(jnp.int32)
    n = idx.shape[0]
    idx_sc = jnp.pad(idx, (0, _IDX_PAD - n), mode="edge")
    idx_tc = jnp.pad(idx, (0, _TC_IDX_PAD - n),
                     constant_values=_D).reshape(1, _TC_IDX_PAD)

    mesh = plsc.VectorSubcoreMesh(
        core_axis_name="c", subcore_axis_name="s",
        num_cores=1, num_subcores=1)
    build_mask = functools.partial(
        pl.kernel,
        mesh=mesh,
        out_type=jax.ShapeDtypeStruct((_D,), jnp.float32),
        scratch_types=[
            pltpu.VMEM((_IDX_PAD,), jnp.int32),
            pltpu.VMEM((_D,), jnp.float32),
            pltpu.SemaphoreType.DMA,
        ],
        compiler_params=pltpu.CompilerParams(needs_layout_passes=False),
    )(_sc_mask_body)
    sc_mask = build_mask(idx_sc).reshape(1, _D)

    out_top = pl.pallas_call(
        _tc_top_body,
        grid=(_TOP_BLKS,),
        in_specs=[
            pl.BlockSpec((1, _TC_IDX_PAD), lambda i: (0, 0)),
            pl.BlockSpec((_BLOCK_ROWS, _D), lambda i: (i, 0)),
        ],
        out_specs=pl.BlockSpec((_BLOCK_ROWS, _D), lambda i: (i, 0)),
        out_shape=jax.ShapeDtypeStruct((_B, _D), jnp.float32),
        scratch_shapes=[pltpu.VMEM((1, _D), jnp.float32)],
        compiler_params=pltpu.CompilerParams(
            dimension_semantics=("arbitrary",),
        ),
    )(idx_tc, x)

    return pl.pallas_call(
        _tc_bot_body,
        grid=(_BOT_BLKS,),
        in_specs=[
            pl.BlockSpec((1, _D), lambda i: (0, 0)),
            pl.BlockSpec((_BLOCK_ROWS, _D), lambda i: (i + _TOP_BLKS, 0)),
            pl.BlockSpec(memory_space=pltpu.MemorySpace.HBM),
        ],
        out_specs=pl.BlockSpec((_BLOCK_ROWS, _D), lambda i: (i + _TOP_BLKS, 0)),
        out_shape=jax.ShapeDtypeStruct((_B, _D), jnp.float32),
        input_output_aliases={2: 0},
        compiler_params=pltpu.CompilerParams(
            dimension_semantics=("arbitrary",),
        ),
    )(sc_mask, x, out_top)


# speedup vs baseline: 4.3790x; 1.0027x over previous
import functools

import jax
import jax.numpy as jnp
from jax import lax
from jax.experimental import pallas as pl
from jax.experimental.pallas import tpu as pltpu
from jax.experimental.pallas import tpu_sc as plsc

_B, _D = 16384, 4096
_BLOCK_ROWS = 928
_TOP_BLKS = 9
_BOT_BLKS = 9
_IDX_PAD = 416
_NJ = _IDX_PAD // 16
_FILL = _D // 16
_TC_IDX_PAD = 512


def _sc_mask_body(idx_hbm, mask_hbm, idx_v, mask_v, sem):
    c = lax.axis_index("c")
    s = lax.axis_index("s")
    wid = s * 2 + c

    @pl.when(wid == 0)
    def _():
        pltpu.sync_copy(idx_hbm, idx_v)
        ones = jnp.ones((16,), jnp.float32)
        zeros = jnp.zeros((16,), jnp.float32)

        def fill_body(i, carry):
            mask_v[pl.ds(i * 16, 16)] = ones
            return carry

        lax.fori_loop(0, _FILL, fill_body, 0)

        def j_body(j, carry):
            colv = idx_v[pl.ds(j * 16, 16)]
            plsc.store_scatter(mask_v, [colv], zeros)
            return carry

        lax.fori_loop(0, _NJ, j_body, 0)
        pltpu.sync_copy(mask_v, mask_hbm)


def _tc_top_body(idx_ref, x_ref, o_ref, mask_ref):
    @pl.when(pl.program_id(0) == 0)
    def _():
        cols = jax.lax.broadcasted_iota(jnp.int32, (1, _D), 1)
        idx = idx_ref[...].reshape(_TC_IDX_PAD, 1)
        hit = jnp.any(idx == cols, axis=0, keepdims=True)
        mask_ref[...] = jnp.where(hit, 0.0, 1.0)

    o_ref[...] = x_ref[...] * mask_ref[...]


def _tc_bot_body(mask_ref, x_ref, prev_ref, o_ref):
    del prev_ref
    o_ref[...] = x_ref[...] * mask_ref[...]


def kernel(x, mask_indices):
    idx = mask_indices.astype(jnp.int32)
    n = idx.shape[0]
    idx_sc = jnp.pad(idx, (0, _IDX_PAD - n), mode="edge")
    idx_tc = jnp.pad(idx, (0, _TC_IDX_PAD - n),
                     constant_values=_D).reshape(1, _TC_IDX_PAD)

    mesh = plsc.VectorSubcoreMesh(
        core_axis_name="c", subcore_axis_name="s",
        num_cores=1, num_subcores=1)
    build_mask = functools.partial(
        pl.kernel,
        mesh=mesh,
        out_type=jax.ShapeDtypeStruct((_D,), jnp.float32),
        scratch_types=[
            pltpu.VMEM((_IDX_PAD,), jnp.int32),
            pltpu.VMEM((_D,), jnp.float32),
            pltpu.SemaphoreType.DMA,
        ],
        compiler_params=pltpu.CompilerParams(needs_layout_passes=False),
    )(_sc_mask_body)
    sc_mask = build_mask(idx_sc).reshape(1, _D)

    out_top = pl.pallas_call(
        _tc_top_body,
        grid=(_TOP_BLKS,),
        in_specs=[
            pl.BlockSpec((1, _TC_IDX_PAD), lambda i: (0, 0)),
            pl.BlockSpec((_BLOCK_ROWS, _D), lambda i: (i, 0)),
        ],
        out_specs=pl.BlockSpec((_BLOCK_ROWS, _D), lambda i: (i, 0)),
        out_shape=jax.ShapeDtypeStruct((_B, _D), jnp.float32),
        scratch_shapes=[pltpu.VMEM((1, _D), jnp.float32)],
        compiler_params=pltpu.CompilerParams(
            dimension_semantics=("arbitrary",),
        ),
    )(idx_tc, x)

    return pl.pallas_call(
        _tc_bot_body,
        grid=(_BOT_BLKS,),
        in_specs=[
            pl.BlockSpec((1, _D), lambda i: (0, 0)),
            pl.BlockSpec((_BLOCK_ROWS, _D), lambda i: (i + _TOP_BLKS, 0)),
            pl.BlockSpec(memory_space=pltpu.MemorySpace.HBM),
        ],
        out_specs=pl.BlockSpec((_BLOCK_ROWS, _D), lambda i: (i + _TOP_BLKS, 0)),
        out_shape=jax.ShapeDtypeStruct((_B, _D), jnp.float32),
        input_output_aliases={2: 0},
        compiler_params=pltpu.CompilerParams(
            dimension_semantics=("arbitrary",),
        ),
    )(sc_mask, x, out_top)
